# baseline (speedup 1.0000x reference)
import functools

import jax
import jax.numpy as jnp
from jax.experimental import pallas as pl
from jax.experimental.pallas import tpu as pltpu

B, L, N, D, H, HD = 16, 512, 4096, 256, 4, 64
NC, NR = 18, 6
W = L + 8


def _body(cu_ref, flat_ref, xyz_ref, wout_ref, bout_ref, wpos_ref, bpos_ref,
          wq_ref, wk_ref, wv_ref, wo_ref, wcls_ref, bcls_ref, wreg_ref,
          breg_ref, coords_ref, classes_ref):
    b = pl.program_id(0)
    s = cu_ref[b]
    e = cu_ref[b + 1]
    ln = e - s

    bpos = bpos_ref[:]

    def embed(f, x):
        out = jnp.maximum(
            jax.lax.dot(f.astype(jnp.bfloat16), wout_ref[:, :],
                        preferred_element_type=jnp.float32) + bout_ref[:],
            0.0)
        pe = jax.lax.dot(x.astype(jnp.bfloat16), wpos_ref[:, :],
                         preferred_element_type=jnp.float32)
        return out + pe + bpos

    base = pl.multiple_of(jnp.minimum((s // 8) * 8, N - W), 8)
    d = s - base
    h_win = embed(flat_ref[pl.ds(base, W), :],
                  xyz_ref[pl.ds(base, W), :])
    h_roll = pltpu.roll(h_win, W - d, axis=0)[:L, :]

    row = jax.lax.broadcasted_iota(jnp.int32, (L, 1), 0)
    lcap = jnp.minimum(ln, L)
    valid = row < lcap

    last = jnp.maximum(e - 1, 0)
    l_al = pl.multiple_of((last // 8) * 8, 8)
    sel = jax.lax.broadcasted_iota(jnp.int32, (8, 1), 0) == (last - l_al)
    h8 = embed(flat_ref[pl.ds(l_al, 8), :], xyz_ref[pl.ds(l_al, 8), :])
    h_last = jnp.sum(jnp.where(sel, h8, 0.0), axis=0, keepdims=True)
    repl = jnp.logical_and(ln > L, row == (L - 1))
    h = jnp.where(repl, h_last, jnp.where(valid, h_roll, bpos))

    hb = h.astype(jnp.bfloat16)
    q = jax.lax.dot(hb, wq_ref[:, :],
                    preferred_element_type=jnp.float32) * 0.125
    k = jax.lax.dot(hb, wk_ref[:, :], preferred_element_type=jnp.float32)
    v = jax.lax.dot(hb, wv_ref[:, :], preferred_element_type=jnp.float32)

    ones_col = jnp.ones((L, 1), jnp.float32)
    kbias = jnp.where(valid, 0.0, -40.0)
    heads = []
    for hh in range(H):
        sl = slice(hh * HD, (hh + 1) * HD)
        qh = jnp.concatenate([q[:, sl], ones_col], axis=1)
        kh = jnp.concatenate([k[:, sl], kbias], axis=1)
        vh = jnp.concatenate([v[:, sl], ones_col], axis=1)
        lg = jax.lax.dot_general(qh, kh, (((1,), (1,)), ((), ())),
                                 preferred_element_type=jnp.float32)
        p = jnp.exp(lg)
        pv = jax.lax.dot(p, vh, preferred_element_type=jnp.float32)
        heads.append(pv[:, :HD] / pv[:, HD:HD + 1])
    ao = jnp.concatenate(heads, axis=-1).astype(jnp.bfloat16)
    box = h + jax.lax.dot(ao, wo_ref[:, :], preferred_element_type=jnp.float32)

    coords_ref[0] = jax.lax.dot(
        box, wreg_ref[:, :], preferred_element_type=jnp.float32) + breg_ref[:]
    classes_ref[0] = jax.lax.dot(
        box, wcls_ref[:, :], preferred_element_type=jnp.float32) + bcls_ref[:]


@functools.partial(jax.jit, static_argnames=("interpret",))
def _run(flat, xyz, cu, W_out, b_out, W_pos, b_pos, Wq, Wk, Wv, Wo,
         W_cls, b_cls, W_reg, b_reg, interpret=False):
    full = lambda shp: pl.BlockSpec(shp, lambda b: (0,) * len(shp))
    out_specs = (
        pl.BlockSpec((1, L, NR), lambda b: (b, 0, 0)),
        pl.BlockSpec((1, L, NC), lambda b: (b, 0, 0)),
    )
    in_specs = [
        pl.BlockSpec(memory_space=pltpu.SMEM),
        full((N, D)),
        full((N, 3)),
        full((D, D)), full((D,)),
        full((3, D)), full((D,)),
        full((D, D)), full((D, D)), full((D, D)), full((D, D)),
        full((D, NC)), full((NC,)),
        full((D, NR)), full((NR,)),
    ]
    coords, classes = pl.pallas_call(
        _body,
        grid=(B,),
        in_specs=in_specs,
        out_specs=out_specs,
        out_shape=(
            jax.ShapeDtypeStruct((B, L, NR), jnp.float32),
            jax.ShapeDtypeStruct((B, L, NC), jnp.float32),
        ),
        interpret=interpret,
    )(cu, flat, xyz,
      W_out.astype(jnp.bfloat16), b_out, W_pos.astype(jnp.bfloat16), b_pos,
      Wq.astype(jnp.bfloat16), Wk.astype(jnp.bfloat16),
      Wv.astype(jnp.bfloat16), Wo.astype(jnp.bfloat16),
      W_cls, b_cls, W_reg, b_reg)
    return coords, classes


def kernel(flat, xyz, keys, cu_seqlens, W_out, b_out, W_pos, b_pos,
           Wq, Wk, Wv, Wo, W_cls, b_cls, W_reg, b_reg):
    del keys
    return _run(flat, xyz, cu_seqlens.astype(jnp.int32), W_out, b_out,
                W_pos, b_pos, Wq, Wk, Wv, Wo, W_cls, b_cls, W_reg, b_reg)


# speedup vs baseline: 1.1274x; 1.1274x over previous
import functools

import jax
import jax.numpy as jnp
from jax.experimental import pallas as pl
from jax.experimental.pallas import tpu as pltpu

B, L, N, D, H, HD = 16, 512, 4096, 256, 4, 64
NC, NR = 18, 6
W = L + 8


def _body(cu_ref, flat_ref, xyz_ref, wout_ref, bout_ref, wpos_ref, bpos_ref,
          wq_ref, wk_ref, wv_ref, wo_ref, wcls_ref, bcls_ref, wreg_ref,
          breg_ref, coords_ref, classes_ref, hroll_ref):
    b = pl.program_id(0)
    s = cu_ref[b]
    e = cu_ref[b + 1]
    ln = e - s

    bpos = bpos_ref[:]

    def embed(f, x):
        out = jnp.maximum(
            jax.lax.dot(f, wout_ref[:, :],
                        preferred_element_type=jnp.float32) + bout_ref[:],
            0.0)
        pe = jax.lax.dot(x, wpos_ref[:, :], preferred_element_type=jnp.float32)
        return out + pe + bpos

    base = pl.multiple_of(jnp.minimum((s // 8) * 8, N - W), 8)
    d = s - base
    h_win = embed(flat_ref[pl.ds(base, W), :],
                  xyz_ref[pl.ds(base, W), :])
    for dd in range(8):
        @pl.when(d == dd)
        def _(dd=dd):
            hroll_ref[...] = pltpu.roll(h_win, W - dd, axis=0)[:L, :]
    @pl.when(d >= 8)
    def _():
        hroll_ref[...] = pltpu.roll(h_win, W - d, axis=0)[:L, :]
    h_roll = hroll_ref[...]

    row = jax.lax.broadcasted_iota(jnp.int32, (L, 1), 0)
    lcap = jnp.minimum(ln, L)
    valid = row < lcap

    last = jnp.maximum(e - 1, 0)
    l_al = pl.multiple_of((last // 8) * 8, 8)
    sel = jax.lax.broadcasted_iota(jnp.int32, (8, 1), 0) == (last - l_al)
    h8 = embed(flat_ref[pl.ds(l_al, 8), :], xyz_ref[pl.ds(l_al, 8), :])
    h_last = jnp.sum(jnp.where(sel, h8, 0.0), axis=0, keepdims=True)
    repl = jnp.logical_and(ln > L, row == (L - 1))
    h = jnp.where(repl, h_last, jnp.where(valid, h_roll, bpos))

    q = jax.lax.dot(h, wq_ref[:, :],
                    preferred_element_type=jnp.float32) * 0.125
    k = jax.lax.dot(h, wk_ref[:, :], preferred_element_type=jnp.float32)
    v = jax.lax.dot(h, wv_ref[:, :], preferred_element_type=jnp.float32)

    ones_col = jnp.ones((L, 1), jnp.float32)
    kbias = jnp.where(valid, 0.0, -40.0)
    heads = []
    for hh in range(H):
        sl = slice(hh * HD, (hh + 1) * HD)
        qh = jnp.concatenate([q[:, sl], ones_col], axis=1)
        kh = jnp.concatenate([k[:, sl], kbias], axis=1)
        vh = jnp.concatenate([v[:, sl], ones_col], axis=1)
        lg = jax.lax.dot_general(qh, kh, (((1,), (1,)), ((), ())),
                                 preferred_element_type=jnp.float32)
        p = jnp.exp(lg)
        pv = jax.lax.dot(p, vh, preferred_element_type=jnp.float32)
        heads.append(pv[:, :HD] / pv[:, HD:HD + 1])
    ao = jnp.concatenate(heads, axis=-1)
    box = h + jax.lax.dot(ao, wo_ref[:, :], preferred_element_type=jnp.float32)

    coords_ref[0] = jax.lax.dot(
        box, wreg_ref[:, :], preferred_element_type=jnp.float32) + breg_ref[:]
    classes_ref[0] = jax.lax.dot(
        box, wcls_ref[:, :], preferred_element_type=jnp.float32) + bcls_ref[:]


@functools.partial(jax.jit, static_argnames=("interpret",))
def _run(flat, xyz, cu, W_out, b_out, W_pos, b_pos, Wq, Wk, Wv, Wo,
         W_cls, b_cls, W_reg, b_reg, interpret=False):
    full = lambda shp: pl.BlockSpec(shp, lambda b: (0,) * len(shp))
    out_specs = (
        pl.BlockSpec((1, L, NR), lambda b: (b, 0, 0)),
        pl.BlockSpec((1, L, NC), lambda b: (b, 0, 0)),
    )
    in_specs = [
        pl.BlockSpec(memory_space=pltpu.SMEM),
        full((N, D)),
        full((N, 3)),
        full((D, D)), full((D,)),
        full((3, D)), full((D,)),
        full((D, D)), full((D, D)), full((D, D)), full((D, D)),
        full((D, NC)), full((NC,)),
        full((D, NR)), full((NR,)),
    ]
    coords, classes = pl.pallas_call(
        _body,
        grid=(B,),
        in_specs=in_specs,
        out_specs=out_specs,
        out_shape=(
            jax.ShapeDtypeStruct((B, L, NR), jnp.float32),
            jax.ShapeDtypeStruct((B, L, NC), jnp.float32),
        ),
        scratch_shapes=[pltpu.VMEM((L, D), jnp.float32)],
        interpret=interpret,
    )(cu, flat, xyz,
      W_out, b_out, W_pos, b_pos,
      Wq, Wk, Wv, Wo, W_cls, b_cls, W_reg, b_reg)
    return coords, classes


def kernel(flat, xyz, keys, cu_seqlens, W_out, b_out, W_pos, b_pos,
           Wq, Wk, Wv, Wo, W_cls, b_cls, W_reg, b_reg):
    del keys
    return _run(flat, xyz, cu_seqlens.astype(jnp.int32), W_out, b_out,
                W_pos, b_pos, Wq, Wk, Wv, Wo, W_cls, b_cls, W_reg, b_reg)


# speedup vs baseline: 1.1493x; 1.0194x over previous
import functools

import jax
import jax.numpy as jnp
from jax.experimental import pallas as pl
from jax.experimental.pallas import tpu as pltpu

B, L, N, D, H, HD = 16, 512, 4096, 256, 4, 64
NC, NR = 18, 6
W = L + 8


def _body(cu_ref, flat_ref, xyz_ref, wout_ref, bout_ref, wpos_ref, bpos_ref,
          wq_ref, wk_ref, wv_ref, wo_ref, wcls_ref, bcls_ref, wreg_ref,
          breg_ref, coords_ref, classes_ref):
    b = pl.program_id(0)
    s = cu_ref[b]
    e = cu_ref[b + 1]
    ln = e - s

    bpos = bpos_ref[:]

    def embed(f, x):
        out = jnp.maximum(
            jax.lax.dot(f, wout_ref[:, :],
                        preferred_element_type=jnp.float32) + bout_ref[:],
            0.0)
        pe = jax.lax.dot(x, wpos_ref[:, :], preferred_element_type=jnp.float32)
        return out + pe + bpos

    base = pl.multiple_of(jnp.minimum((s // 8) * 8, N - W), 8)
    d = s - base
    h_win = embed(flat_ref[pl.ds(base, W), :],
                  xyz_ref[pl.ds(base, W), :])
    h_roll = pltpu.roll(h_win, W - d, axis=0)[:L, :]

    row = jax.lax.broadcasted_iota(jnp.int32, (L, 1), 0)
    lcap = jnp.minimum(ln, L)
    valid = row < lcap

    last = jnp.maximum(e - 1, 0)
    l_al = pl.multiple_of((last // 8) * 8, 8)
    sel = jax.lax.broadcasted_iota(jnp.int32, (8, 1), 0) == (last - l_al)
    h8 = embed(flat_ref[pl.ds(l_al, 8), :], xyz_ref[pl.ds(l_al, 8), :])
    h_last = jnp.sum(jnp.where(sel, h8, 0.0), axis=0, keepdims=True)
    repl = jnp.logical_and(ln > L, row == (L - 1))
    h = jnp.where(repl, h_last, jnp.where(valid, h_roll, bpos))

    q = jax.lax.dot(h, wq_ref[:, :],
                    preferred_element_type=jnp.float32) * 0.125
    k = jax.lax.dot(h, wk_ref[:, :], preferred_element_type=jnp.float32)
    v = jax.lax.dot(h, wv_ref[:, :], preferred_element_type=jnp.float32)

    ones_col = jnp.ones((L, 1), jnp.float32)
    kbias = jnp.where(valid, 0.0, -40.0)
    heads = []
    for hh in range(H):
        sl = slice(hh * HD, (hh + 1) * HD)
        qh = jnp.concatenate([q[:, sl], ones_col], axis=1)
        kh = jnp.concatenate([k[:, sl], kbias], axis=1)
        vh = jnp.concatenate([v[:, sl], ones_col], axis=1)
        lg = jax.lax.dot_general(qh, kh, (((1,), (1,)), ((), ())),
                                 preferred_element_type=jnp.float32)
        p = jnp.exp(lg)
        pv = jax.lax.dot(p, vh, preferred_element_type=jnp.float32)
        heads.append(pv[:, :HD] / pv[:, HD:HD + 1])
    ao = jnp.concatenate(heads, axis=-1)
    box = h + jax.lax.dot(ao, wo_ref[:, :], preferred_element_type=jnp.float32)

    coords_ref[0] = jax.lax.dot(
        box, wreg_ref[:, :], preferred_element_type=jnp.float32) + breg_ref[:]
    classes_ref[0] = jax.lax.dot(
        box, wcls_ref[:, :], preferred_element_type=jnp.float32) + bcls_ref[:]


@functools.partial(jax.jit, static_argnames=("interpret",))
def _run(flat, xyz, cu, W_out, b_out, W_pos, b_pos, Wq, Wk, Wv, Wo,
         W_cls, b_cls, W_reg, b_reg, interpret=False):
    full = lambda shp: pl.BlockSpec(shp, lambda b: (0,) * len(shp))
    out_specs = (
        pl.BlockSpec((1, L, NR), lambda b: (b, 0, 0)),
        pl.BlockSpec((1, L, NC), lambda b: (b, 0, 0)),
    )
    in_specs = [
        pl.BlockSpec(memory_space=pltpu.SMEM),
        full((N, D)),
        full((N, 3)),
        full((D, D)), full((D,)),
        full((3, D)), full((D,)),
        full((D, D)), full((D, D)), full((D, D)), full((D, D)),
        full((D, NC)), full((NC,)),
        full((D, NR)), full((NR,)),
    ]
    coords, classes = pl.pallas_call(
        _body,
        grid=(B,),
        in_specs=in_specs,
        out_specs=out_specs,
        out_shape=(
            jax.ShapeDtypeStruct((B, L, NR), jnp.float32),
            jax.ShapeDtypeStruct((B, L, NC), jnp.float32),
        ),
        interpret=interpret,
    )(cu, flat, xyz,
      W_out, b_out, W_pos, b_pos,
      Wq, Wk, Wv, Wo, W_cls, b_cls, W_reg, b_reg)
    return coords, classes


def kernel(flat, xyz, keys, cu_seqlens, W_out, b_out, W_pos, b_pos,
           Wq, Wk, Wv, Wo, W_cls, b_cls, W_reg, b_reg):
    del keys
    return _run(flat, xyz, cu_seqlens.astype(jnp.int32), W_out, b_out,
                W_pos, b_pos, Wq, Wk, Wv, Wo, W_cls, b_cls, W_reg, b_reg)
